# Initial kernel scaffold; baseline (speedup 1.0000x reference)
import jax
import jax.numpy as jnp
from jax.experimental import pallas as pl


def kernel(bf, h, knn, h_edge, cutoff, W1, b1, W2, b2, W3, b3, W4, b4):
    raise NotImplementedError("write your pallas kernel here")


# speedup vs baseline: 1.4516x; 1.4516x over previous
import functools

import jax
import jax.numpy as jnp
import numpy as np
from jax import lax
from jax.experimental import pallas as pl
from jax.experimental.pallas import tpu as pltpu
from jax.experimental.pallas import tpu_sc as plsc

N = 10000
K = 32
RADIAL = 256
HIDDEN = 128
OUT = 64
E = N * K
LOG2 = float(np.log(2.0))


_CH = 80


def _sc_gather(table, idx):
    info = plsc.get_sparse_core_info()
    _NC, _NS = info.num_cores, info.num_subcores
    _E_PER_W = E // (_NC * _NS)
    _NCHUNK = _E_PER_W // _CH
    mesh = plsc.VectorSubcoreMesh(core_axis_name="c", subcore_axis_name="s")

    @functools.partial(
        pl.kernel,
        mesh=mesh,
        out_type=jax.ShapeDtypeStruct((E, OUT), jnp.float32),
        scratch_types=[
            pltpu.VMEM((_E_PER_W,), jnp.int32),
            pltpu.VMEM((_CH, OUT), jnp.float32),
            pltpu.SemaphoreType.DMA,
        ],
        compiler_params=pltpu.CompilerParams(use_tc_tiling_on_sc=False),
    )
    def gather_kernel(table_hbm, idx_hbm, out_hbm, idx_v, rows_v, sem):
        wid = lax.axis_index("s") * _NC + lax.axis_index("c")
        base = wid * _E_PER_W
        pltpu.sync_copy(idx_hbm.at[pl.ds(base, _E_PER_W)], idx_v)

        def body(c, carry):
            off = c * _CH
            pltpu.async_copy(
                table_hbm.at[idx_v.at[pl.ds(off, _CH)]], rows_v, sem
            ).wait()
            pltpu.sync_copy(rows_v, out_hbm.at[pl.ds(base + off, _CH)])
            return carry

        lax.fori_loop(0, _NCHUNK, body, 0)

    return gather_kernel(table, idx)


_BN = 80
_EB = _BN * K
_GRID = N // _BN


def _ssp(x):
    return jnp.maximum(x, 0.0) + jnp.log1p(jnp.exp(-jnp.abs(x))) - LOG2


def _tc_body(bf_ref, hs_ref, he_ref, co_ref,
             w1_ref, b1_ref, w2_ref, b2_ref, w3_ref, b3_ref, w4_ref, b4_ref,
             out_ref):
    x = bf_ref[...]
    x = _ssp(jnp.dot(x, w1_ref[...],
                     preferred_element_type=jnp.float32) + b1_ref[...])
    x = _ssp(jnp.dot(x, w2_ref[...],
                     preferred_element_type=jnp.float32) + b2_ref[...])
    v = x * hs_ref[...] * he_ref[...] * co_ref[...]
    m = jnp.sum(v.reshape(_BN, K, OUT), axis=1)
    m = _ssp(jnp.dot(m, w3_ref[...],
                     preferred_element_type=jnp.float32) + b3_ref[...])
    out_ref[...] = _ssp(jnp.dot(m, w4_ref[...],
                                preferred_element_type=jnp.float32) + b4_ref[...])


def _tc_conv(bf2, h_src, he2, co2, W1, b1, W2, b2, W3, b3, W4, b4):
    edge_spec = lambda w: pl.BlockSpec((_EB, w), lambda i: (i, 0))
    full_spec = lambda a: pl.BlockSpec(a.shape, lambda i: (0,) * a.ndim)
    return pl.pallas_call(
        _tc_body,
        grid=(_GRID,),
        in_specs=[
            edge_spec(RADIAL),
            edge_spec(OUT),
            edge_spec(OUT),
            edge_spec(1),
            full_spec(W1), full_spec(b1),
            full_spec(W2), full_spec(b2),
            full_spec(W3), full_spec(b3),
            full_spec(W4), full_spec(b4),
        ],
        out_specs=pl.BlockSpec((_BN, OUT), lambda i: (i, 0)),
        out_shape=jax.ShapeDtypeStruct((N, OUT), jnp.float32),
        compiler_params=pltpu.CompilerParams(
            dimension_semantics=("arbitrary",),
        ),
    )(bf2, h_src, he2, co2, W1, b1, W2, b2, W3, b3, W4, b4)


def kernel(bf, h, knn, h_edge, cutoff, W1, b1, W2, b2, W3, b3, W4, b4):
    idx = knn.reshape(-1).astype(jnp.int32)
    h_src = _sc_gather(h, idx)
    bf2 = bf.reshape(E, RADIAL)
    he2 = h_edge.reshape(E, OUT)
    co2 = cutoff.reshape(E, 1)
    return _tc_conv(bf2, h_src, he2, co2,
                    W1, b1.reshape(1, HIDDEN), W2, b2.reshape(1, OUT),
                    W3, b3.reshape(1, OUT), W4, b4.reshape(1, OUT))
